# baseline (speedup 1.0000x reference)
import dataclasses
import functools

import jax
import jax.numpy as jnp
from jax import lax
from jax.experimental import pallas as pl
from jax.experimental.pallas import tpu as pltpu
from jax.experimental.pallas import tpu_sc as plsc

_NUM_CORES = 2
_NUM_SUBCORES = 16
_LANES = 16
_NW = _NUM_CORES * _NUM_SUBCORES
_TBL_PAD = 128


def _sc_compiler_params():
    cp = pltpu.CompilerParams()
    if "needs_layout_passes" in pltpu.CompilerParams.__dataclass_fields__:
        cp = dataclasses.replace(cp, needs_layout_passes=False)
    return cp


def _embed_sc(tbl, inputs):
    b, l = inputs.shape
    rows = b // _NW
    offs = list(range(0, l - _LANES + 1, _LANES))
    if offs[-1] != l - _LANES:
        offs.append(l - _LANES)

    mesh = plsc.VectorSubcoreMesh(
        core_axis_name="c", subcore_axis_name="s",
        num_cores=_NUM_CORES, num_subcores=_NUM_SUBCORES,
    )

    @functools.partial(
        pl.kernel,
        out_type=jax.ShapeDtypeStruct((l, b), jnp.float32),
        mesh=mesh,
        scratch_types=[
            pltpu.VMEM((_TBL_PAD,), jnp.float32),
            pltpu.VMEM((rows, l), jnp.int32),
            pltpu.VMEM((l, rows), jnp.float32),
        ],
        compiler_params=_sc_compiler_params(),
    )
    def body(tbl_hbm, idx_hbm, out_hbm, tbl_v, idx_v, out_v):
        wid = lax.axis_index("c") * _NUM_SUBCORES + lax.axis_index("s")
        r0 = wid * rows
        pltpu.sync_copy(tbl_hbm, tbl_v)
        pltpu.sync_copy(idx_hbm.at[pl.ds(r0, rows)], idx_v)

        lane_iota = lax.iota(jnp.int32, _LANES)

        @plsc.parallel_loop(0, rows, step=1, unroll=2)
        def _(r):
            r_vec = jnp.full((_LANES,), r, jnp.int32)
            for c in offs:
                idx = idx_v[r, pl.ds(c, _LANES)]
                val = plsc.load_gather(tbl_v, [idx])
                plsc.store_scatter(out_v, [lane_iota + c, r_vec], val)

        pltpu.sync_copy(out_v, out_hbm.at[:, pl.ds(r0, rows)])

    return body(tbl, inputs)


def kernel(inputs, z_weights):
    tbl = jnp.pad(z_weights[:, 0], (0, _TBL_PAD - z_weights.shape[0]))
    out_t = _embed_sc(tbl, inputs.astype(jnp.int32))
    return out_t.T[:, :, None]


# speedup vs baseline: 1.2186x; 1.2186x over previous
import dataclasses
import functools

import jax
import jax.numpy as jnp
from jax import lax
from jax.experimental import pallas as pl
from jax.experimental.pallas import tpu as pltpu
from jax.experimental.pallas import tpu_sc as plsc

_NUM_CORES = 2
_NUM_SUBCORES = 16
_LANES = 16
_NW = _NUM_CORES * _NUM_SUBCORES
_TBL_PAD = 128


def _sc_compiler_params():
    cp = pltpu.CompilerParams()
    if "needs_layout_passes" in pltpu.CompilerParams.__dataclass_fields__:
        cp = dataclasses.replace(cp, needs_layout_passes=False)
    return cp


def _embed_sc(tbl, inputs):
    b, l = inputs.shape
    rows = b // _NW
    offs = list(range(0, l - _LANES + 1, _LANES))
    if offs[-1] != l - _LANES:
        offs.append(l - _LANES)

    mesh = plsc.VectorSubcoreMesh(
        core_axis_name="c", subcore_axis_name="s",
        num_cores=_NUM_CORES, num_subcores=_NUM_SUBCORES,
    )

    blk_rows = 32

    @functools.partial(
        pl.kernel,
        out_type=jax.ShapeDtypeStruct((b, l), jnp.float32),
        mesh=mesh,
        scratch_types=[
            pltpu.VMEM((_TBL_PAD,), jnp.float32),
        ],
        compiler_params=_sc_compiler_params(),
    )
    def body(tbl_hbm, idx_hbm, out_hbm, tbl_v):
        pltpu.sync_copy(tbl_hbm, tbl_v)

        def block_body(idx_v, out_v):
            @plsc.parallel_loop(0, blk_rows, step=1, unroll=2)
            def _(r):
                for c in offs:
                    idx = idx_v[r, pl.ds(c, _LANES)]
                    out_v[r, pl.ds(c, _LANES)] = plsc.load_gather(tbl_v, [idx])

        pltpu.emit_pipeline(
            block_body,
            grid=(b // blk_rows,),
            in_specs=[pl.BlockSpec((blk_rows, l), lambda i: (i, 0))],
            out_specs=[pl.BlockSpec((blk_rows, l), lambda i: (i, 0))],
            core_axis_name=("c", "s"),
            dimension_semantics=(pltpu.PARALLEL,),
        )(idx_hbm, out_hbm)

    return body(tbl, inputs)


def kernel(inputs, z_weights):
    tbl = jnp.pad(z_weights[:, 0], (0, _TBL_PAD - z_weights.shape[0]))
    return _embed_sc(tbl, inputs.astype(jnp.int32))[..., None]
